# baseline (speedup 1.0000x reference)
import functools

import jax
import jax.numpy as jnp
from jax import lax
from jax.experimental import pallas as pl
from jax.experimental.pallas import tpu as pltpu
from jax.experimental.pallas import tpu_sc as plsc

V = 1000000
D = 16
B = 4096
L = 50
FC1_OUT = 128
NUM_CLASSES = 10

NC, NS = 2, 16
NW = NC * NS

TCOLS = V // 128
COLS_W = 244
CB = 4
NSTEP = COLS_W // CB
AROWS = CB * 128
VTAIL = V % 128

BPW = B // NW
CH = 64
NCHUNK = BPW // CH
G = 128
EL = CH * L
NG = EL // G
LP = 64


def _sc_relayout(E0t, E1t, E2t, E3t, tails):
    mesh = plsc.VectorSubcoreMesh(core_axis_name="c", subcore_axis_name="s")
    oshape = jax.ShapeDtypeStruct((V // 8, 128), jnp.float32)

    @functools.partial(
        pl.kernel,
        out_type=(oshape,) * 4,
        mesh=mesh,
        scratch_types=[
            pltpu.VMEM((2, 8, AROWS), jnp.float32),
            pltpu.VMEM((2, 8, AROWS), jnp.float32),
            pltpu.VMEM((AROWS // 8, 128), jnp.float32),
            pltpu.VMEM((AROWS // 8, 128), jnp.float32),
            pltpu.SemaphoreType.DMA,
            pltpu.SemaphoreType.DMA,
            pltpu.SemaphoreType.DMA,
            pltpu.SemaphoreType.DMA,
        ],
        compiler_params=pltpu.CompilerParams(
            use_tc_tiling_on_sc=True, needs_layout_passes=False
        ),
    )
    def k(t0, t1, t2, t3, tl, o0, o1, o2, o3,
          ta0, ta1, la0, la1, ins0, ins1, outs0, outs1):
        cid = lax.axis_index("c")
        sid = lax.axis_index("s")
        wid = sid * NC + cid
        dd = lax.iota(jnp.int32, 16)
        AV = lax.shift_right_logical(dd, 3)
        SV = lax.bitwise_and(dd, 7)
        ZV = dd * 0
        tas = (ta0, ta1)
        las = (la0, la1)
        inss = (ins0, ins1)
        outss = (outs0, outs1)
        c0 = wid * COLS_W

        def transform(tbuf, lbuf, nrows):
            @plsc.parallel_loop(0, nrows // 8, unroll=8)
            def _(jj):
                for u in range(8):
                    row = plsc.load_gather(tbuf, [AV, SV, ZV + (jj * 8 + u)])
                    lbuf[jj, pl.ds(u * 16, 16)] = row

        for tsrc, out in ((t0, o0), (t1, o1), (t2, o2), (t3, o3)):

            def issue_in(k_, par, tsrc=tsrc):
                pltpu.async_copy(
                    tsrc.at[:, :, pl.ds((c0 + CB * k_) * 128, AROWS)],
                    tas[par], inss[par])

            def wait_in(k_, par, tsrc=tsrc):
                pltpu.make_async_copy(
                    tsrc.at[:, :, pl.ds((c0 + CB * k_) * 128, AROWS)],
                    tas[par], inss[par]).wait()

            def out_desc(k_, par, out=out):
                return pltpu.make_async_copy(
                    las[par],
                    out.at[pl.ds((c0 + CB * k_) * 16, AROWS // 8)],
                    outss[par])

            def step(k_, par):
                wait_in(k_, par)

                @pl.when(k_ < NSTEP - 1)
                def _():
                    issue_in(k_ + 1, 1 - par)

                @pl.when(k_ >= 2)
                def _():
                    out_desc(k_ - 2, par).wait()

                transform(tas[par], las[par], AROWS)
                out_desc(k_, par).start()

            issue_in(0, 0)

            def step2(kk, _):
                step(kk * 2, 0)
                step(kk * 2 + 1, 1)
                return 0

            lax.fori_loop(0, (NSTEP - 1) // 2, step2, 0)
            step(NSTEP - 1, 0)
            out_desc(NSTEP - 2, 1).wait()
            out_desc(NSTEP - 1, 0).wait()

            @pl.when(wid < 4)
            def _(tsrc=tsrc, out=out):
                c = NW * COLS_W + wid
                pltpu.sync_copy(tsrc.at[:, :, pl.ds(c * 128, 128)],
                                ta0.at[:, :, pl.ds(0, 128)])
                transform(ta0, la0, 128)
                pltpu.sync_copy(la0.at[pl.ds(0, 16)],
                                out.at[pl.ds(c * 16, 16)])

        @pl.when(wid == 4)
        def _():
            for f, out in enumerate((o0, o1, o2, o3)):
                pltpu.sync_copy(tl.at[f], la1.at[pl.ds(0, VTAIL // 8)])
                pltpu.sync_copy(la1.at[pl.ds(0, VTAIL // 8)],
                                out.at[pl.ds((V - VTAIL) // 8, VTAIL // 8)])

    return k(E0t, E1t, E2t, E3t, tails)


def _sc_pool(xidx, wpad, T0, T1, T2, T3):
    mesh = plsc.VectorSubcoreMesh(core_axis_name="c", subcore_axis_name="s")

    @functools.partial(
        pl.kernel,
        out_type=jax.ShapeDtypeStruct((4, B, D), jnp.float32),
        mesh=mesh,
        scratch_types=[
            pltpu.VMEM((EL,), jnp.int32),
            pltpu.VMEM((CH, LP), jnp.float32),
            pltpu.VMEM((EL, D), jnp.float32),
            pltpu.VMEM((CH, D), jnp.float32),
            pltpu.SemaphoreType.DMA,
        ],
        compiler_params=pltpu.CompilerParams(use_tc_tiling_on_sc=False),
    )
    def k(ix, w_hbm, t0, t1, t2, t3, out_hbm, idx_v, w_v, rows_v, out_v, sem):
        wid = lax.axis_index("s") * NC + lax.axis_index("c")

        for f, tab in enumerate((t0, t1, t2, t3)):

            def chunk_body(c, carry, f=f, tab=tab):
                b0 = wid * BPW + c * CH
                pltpu.sync_copy(ix.at[f, pl.ds(b0 * L, EL)], idx_v)
                pltpu.sync_copy(w_hbm.at[pl.ds(b0, CH)], w_v)
                cps = [
                    pltpu.async_copy(
                        tab.at[idx_v.at[pl.ds(g * G, G)]],
                        rows_v.at[pl.ds(g * G, G)],
                        sem,
                    )
                    for g in range(NG)
                ]
                for cp in cps:
                    cp.wait()

                def row_body(jj, rc):
                    e0 = jj * L
                    wv = [w_v[jj, pl.ds(16 * kk, 16)] for kk in range(4)]
                    accs = [wv[0][i] * rows_v[e0 + i] for i in range(4)]
                    for l in range(4, L):
                        accs[l % 4] = (
                            accs[l % 4] + wv[l // 16][l % 16] * rows_v[e0 + l]
                        )
                    out_v[jj] = (accs[0] + accs[1]) + (accs[2] + accs[3])
                    return rc

                lax.fori_loop(0, CH, row_body, 0)
                pltpu.sync_copy(out_v, out_hbm.at[f, pl.ds(b0, CH)])
                return carry

            lax.fori_loop(0, NCHUNK, chunk_body, 0)

    return k(xidx, wpad, T0, T1, T2, T3)


def _mlp_body(p_ref, w1_ref, b1_ref, w2_ref, b2_ref, o_ref):
    h = jnp.zeros((p_ref.shape[1], FC1_OUT), jnp.float32)
    for f in range(4):
        h = h + jnp.dot(p_ref[f], w1_ref[f], preferred_element_type=jnp.float32)
    h = jnp.maximum(h + b1_ref[0], 0.0)
    o = jnp.dot(h, w2_ref[...], preferred_element_type=jnp.float32) + b2_ref[0]
    o_ref[...] = o


def _mlp(pooled, W1, b1, W2, b2):
    BLK = 512
    grid = (B // BLK,)
    return pl.pallas_call(
        _mlp_body,
        grid=grid,
        in_specs=[
            pl.BlockSpec((4, BLK, D), lambda i: (0, i, 0)),
            pl.BlockSpec((4, D, FC1_OUT), lambda i: (0, 0, 0)),
            pl.BlockSpec((1, FC1_OUT), lambda i: (0, 0)),
            pl.BlockSpec((FC1_OUT, NUM_CLASSES), lambda i: (0, 0)),
            pl.BlockSpec((1, NUM_CLASSES), lambda i: (0, 0)),
        ],
        out_specs=pl.BlockSpec((BLK, NUM_CLASSES), lambda i: (i, 0)),
        out_shape=jax.ShapeDtypeStruct((B, NUM_CLASSES), jnp.float32),
    )(pooled, W1.reshape(4, D, FC1_OUT), b1.reshape(1, FC1_OUT), W2,
      b2.reshape(1, NUM_CLASSES))


def kernel(x, E0, E1, E2, E3, W1, b1, W2, b2):
    tabs_t = [E.T.reshape(2, 8, V) for E in (E0, E1, E2, E3)]
    tails = jnp.stack(
        [E[V - VTAIL:].reshape(VTAIL // 8, 128) for E in (E0, E1, E2, E3)]
    )
    lin = _sc_relayout(*tabs_t, tails)
    tabs = [t.reshape(V, D) for t in lin]
    xidx = x[:4].reshape(4, B * L)
    wpad = jnp.pad(x[4].astype(jnp.float32), ((0, 0), (0, LP - L)))
    pooled = _sc_pool(xidx, wpad, *tabs)
    return _mlp(pooled, W1, b1, W2, b2)


# speedup vs baseline: 2.0560x; 2.0560x over previous
import functools

import jax
import jax.numpy as jnp
from jax import lax
from jax.experimental import pallas as pl
from jax.experimental.pallas import tpu as pltpu
from jax.experimental.pallas import tpu_sc as plsc

V = 1000000
D = 16
B = 4096
L = 50
FC1_OUT = 128
NUM_CLASSES = 10

NC, NS = 2, 16
NW = NC * NS

TCOLS = V // 128
COLS_W = 244
CB = 4
NSTEP = COLS_W // CB
AROWS = CB * 128
VTAIL = V % 128

BPW = B // NW
CH = 64
NCHUNK = BPW // CH
G = 128
EL = CH * L
NG = EL // G
LP = 64


def _sc_relayout(E0t, E1t, E2t, E3t, tails):
    mesh = plsc.VectorSubcoreMesh(core_axis_name="c", subcore_axis_name="s")
    oshape = jax.ShapeDtypeStruct((V * D,), jnp.float32)

    @functools.partial(
        pl.kernel,
        out_type=(oshape,) * 4,
        mesh=mesh,
        scratch_types=[
            pltpu.VMEM((2, 8, AROWS), jnp.float32),
            pltpu.VMEM((2, 8, AROWS), jnp.float32),
            pltpu.VMEM((AROWS * D,), jnp.float32),
            pltpu.VMEM((AROWS * D,), jnp.float32),
            pltpu.SemaphoreType.DMA,
            pltpu.SemaphoreType.DMA,
            pltpu.SemaphoreType.DMA,
            pltpu.SemaphoreType.DMA,
        ],
        compiler_params=pltpu.CompilerParams(
            use_tc_tiling_on_sc=True, needs_layout_passes=False
        ),
    )
    def k(t0, t1, t2, t3, tl, o0, o1, o2, o3,
          ta0, ta1, la0, la1, ins0, ins1, outs0, outs1):
        cid = lax.axis_index("c")
        sid = lax.axis_index("s")
        wid = sid * NC + cid
        dd = lax.iota(jnp.int32, 16)
        I16 = dd * D
        tas = (ta0, ta1)
        las = (la0, la1)
        inss = (ins0, ins1)
        outss = (outs0, outs1)
        c0 = wid * COLS_W

        def transform(tbuf, lbuf, nrows):
            @plsc.parallel_loop(0, nrows // 16, unroll=4)
            def _(jb):
                j0 = jb * 16
                base = j0 * D
                for a in range(2):
                    for s in range(8):
                        vals = tbuf[a, s, pl.ds(j0, 16)]
                        plsc.store_scatter(
                            lbuf, [I16 + (base + 8 * a + s)], vals)

        for tsrc, out in ((t0, o0), (t1, o1), (t2, o2), (t3, o3)):

            def issue_in(k_, par, tsrc=tsrc):
                pltpu.async_copy(
                    tsrc.at[:, :, pl.ds((c0 + CB * k_) * 128, AROWS)],
                    tas[par], inss[par])

            def wait_in(k_, par, tsrc=tsrc):
                pltpu.make_async_copy(
                    tsrc.at[:, :, pl.ds((c0 + CB * k_) * 128, AROWS)],
                    tas[par], inss[par]).wait()

            def out_desc(k_, par, out=out):
                return pltpu.make_async_copy(
                    las[par],
                    out.at[pl.ds((c0 + CB * k_) * 128 * D, AROWS * D)],
                    outss[par])

            def step(k_, par):
                wait_in(k_, par)

                @pl.when(k_ < NSTEP - 1)
                def _():
                    issue_in(k_ + 1, 1 - par)

                @pl.when(k_ >= 2)
                def _():
                    out_desc(k_ - 2, par).wait()

                transform(tas[par], las[par], AROWS)
                out_desc(k_, par).start()

            issue_in(0, 0)

            def step2(kk, _):
                step(kk * 2, 0)
                step(kk * 2 + 1, 1)
                return 0

            lax.fori_loop(0, (NSTEP - 1) // 2, step2, 0)
            step(NSTEP - 1, 0)
            out_desc(NSTEP - 2, 1).wait()
            out_desc(NSTEP - 1, 0).wait()

            @pl.when(wid < 4)
            def _(tsrc=tsrc, out=out):
                c = NW * COLS_W + wid
                pltpu.sync_copy(tsrc.at[:, :, pl.ds(c * 128, 128)],
                                ta0.at[:, :, pl.ds(0, 128)])
                transform(ta0, la0, 128)
                pltpu.sync_copy(la0.at[pl.ds(0, 128 * D)],
                                out.at[pl.ds(c * 128 * D, 128 * D)])

        @pl.when(wid == 4)
        def _():
            for f, out in enumerate((o0, o1, o2, o3)):
                pltpu.sync_copy(tl.at[f], la1.at[pl.ds(0, VTAIL * D)])
                pltpu.sync_copy(la1.at[pl.ds(0, VTAIL * D)],
                                out.at[pl.ds((V - VTAIL) * D, VTAIL * D)])

    return k(E0t, E1t, E2t, E3t, tails)


def _sc_pool(xidx, wpad, T0, T1, T2, T3):
    mesh = plsc.VectorSubcoreMesh(core_axis_name="c", subcore_axis_name="s")

    @functools.partial(
        pl.kernel,
        out_type=jax.ShapeDtypeStruct((4, B, D), jnp.float32),
        mesh=mesh,
        scratch_types=[
            pltpu.VMEM((EL,), jnp.int32),
            pltpu.VMEM((CH, LP), jnp.float32),
            pltpu.VMEM((EL, D), jnp.float32),
            pltpu.VMEM((CH, D), jnp.float32),
            pltpu.SemaphoreType.DMA,
        ],
        compiler_params=pltpu.CompilerParams(use_tc_tiling_on_sc=False),
    )
    def k(ix, w_hbm, t0, t1, t2, t3, out_hbm, idx_v, w_v, rows_v, out_v, sem):
        wid = lax.axis_index("s") * NC + lax.axis_index("c")

        for f, tab in enumerate((t0, t1, t2, t3)):

            def chunk_body(c, carry, f=f, tab=tab):
                b0 = wid * BPW + c * CH
                pltpu.sync_copy(ix.at[f, pl.ds(b0 * L, EL)], idx_v)
                pltpu.sync_copy(w_hbm.at[pl.ds(b0, CH)], w_v)
                cps = [
                    pltpu.async_copy(
                        tab.at[idx_v.at[pl.ds(g * G, G)]],
                        rows_v.at[pl.ds(g * G, G)],
                        sem,
                    )
                    for g in range(NG)
                ]
                for cp in cps:
                    cp.wait()

                def row_body(jj, rc):
                    e0 = jj * L
                    wv = [w_v[jj, pl.ds(16 * kk, 16)] for kk in range(4)]
                    accs = [wv[0][i] * rows_v[e0 + i] for i in range(4)]
                    for l in range(4, L):
                        accs[l % 4] = (
                            accs[l % 4] + wv[l // 16][l % 16] * rows_v[e0 + l]
                        )
                    out_v[jj] = (accs[0] + accs[1]) + (accs[2] + accs[3])
                    return rc

                lax.fori_loop(0, CH, row_body, 0)
                pltpu.sync_copy(out_v, out_hbm.at[f, pl.ds(b0, CH)])
                return carry

            lax.fori_loop(0, NCHUNK, chunk_body, 0)

    return k(xidx, wpad, T0, T1, T2, T3)


def _mlp_body(p_ref, w1_ref, b1_ref, w2_ref, b2_ref, o_ref):
    h = jnp.zeros((p_ref.shape[1], FC1_OUT), jnp.float32)
    for f in range(4):
        h = h + jnp.dot(p_ref[f], w1_ref[f], preferred_element_type=jnp.float32)
    h = jnp.maximum(h + b1_ref[0], 0.0)
    o = jnp.dot(h, w2_ref[...], preferred_element_type=jnp.float32) + b2_ref[0]
    o_ref[...] = o


def _mlp(pooled, W1, b1, W2, b2):
    BLK = 512
    grid = (B // BLK,)
    return pl.pallas_call(
        _mlp_body,
        grid=grid,
        in_specs=[
            pl.BlockSpec((4, BLK, D), lambda i: (0, i, 0)),
            pl.BlockSpec((4, D, FC1_OUT), lambda i: (0, 0, 0)),
            pl.BlockSpec((1, FC1_OUT), lambda i: (0, 0)),
            pl.BlockSpec((FC1_OUT, NUM_CLASSES), lambda i: (0, 0)),
            pl.BlockSpec((1, NUM_CLASSES), lambda i: (0, 0)),
        ],
        out_specs=pl.BlockSpec((BLK, NUM_CLASSES), lambda i: (i, 0)),
        out_shape=jax.ShapeDtypeStruct((B, NUM_CLASSES), jnp.float32),
    )(pooled, W1.reshape(4, D, FC1_OUT), b1.reshape(1, FC1_OUT), W2,
      b2.reshape(1, NUM_CLASSES))


def kernel(x, E0, E1, E2, E3, W1, b1, W2, b2):
    tabs_t = [E.T.reshape(2, 8, V) for E in (E0, E1, E2, E3)]
    tails = jnp.stack(
        [E[V - VTAIL:].reshape(VTAIL * D) for E in (E0, E1, E2, E3)]
    )
    lin = _sc_relayout(*tabs_t, tails)
    tabs = [t.reshape(V, D) for t in lin]
    xidx = x[:4].reshape(4, B * L)
    wpad = jnp.pad(x[4].astype(jnp.float32), ((0, 0), (0, LP - L)))
    pooled = _sc_pool(xidx, wpad, *tabs)
    return _mlp(pooled, W1, b1, W2, b2)
